# baseline (speedup 1.0000x reference)
import jax
import jax.numpy as jnp
from jax import lax
from jax.experimental import pallas as pl
from jax.experimental.pallas import tpu as pltpu
from jax.experimental.pallas import tpu_sc as plsc

VOCAB = 100000
D = 64
T = 200
B = 1024

NC = 2
NS = 16
NW = NC * NS
ROWS_PER_W = B // NW
NBUF = 2
LANES = 16
EMBED_SCALE = 8.0

SPLITS = ((0, 104), (104, 96))


def _body(idx_hbm, table_hbm, pos_hbm, out_hbm,
          pos_v, idx_all, rows_v, out_v, sem_g0, sem_g1, sem_w0, sem_w1):
    wid = lax.axis_index("s") * NC + lax.axis_index("c")
    base = wid * ROWS_PER_W
    sem_g = (sem_g0, sem_g1)
    sem_w = (sem_w0, sem_w1)

    pltpu.sync_copy(pos_hbm, pos_v)
    pltpu.sync_copy(idx_hbm.at[pl.ds(base * T, ROWS_PER_W * T)], idx_all)

    def gather_half(r, u, h):
        off, ln = SPLITS[h]
        start = pl.multiple_of(r * T, 8) + off
        return pltpu.make_async_copy(
            table_hbm.at[idx_all.at[pl.ds(start, ln)]],
            rows_v.at[u, pl.ds(off, ln)], sem_g[u])

    def writeback(r, u):
        return pltpu.make_async_copy(out_v.at[u], out_hbm.at[base + r], sem_w[u])

    for u in range(NBUF):
        gather_half(u, u, 0).start()
        gather_half(u, u, 1).start()

    def iter_body(it, carry):
        for u in range(NBUF):
            r = it * NBUF + u
            gather_half(r, u, 0).wait()
            gather_half(r, u, 1).wait()

            @pl.when(it > 0)
            def _():
                writeback(r - NBUF, u).wait()

            @plsc.parallel_loop(0, T, 1, unroll=8)
            def _compute(i):
                for c in range(D // LANES):
                    sl = pl.ds(c * LANES, LANES)
                    out_v[u, i, sl] = (
                        rows_v[u, i, sl] * EMBED_SCALE
                        + pos_v[pl.ds(i * D + c * LANES, LANES)])

            @pl.when(it + 1 < ROWS_PER_W // NBUF)
            def _():
                gather_half(r + NBUF, u, 0).start()
                gather_half(r + NBUF, u, 1).start()

            writeback(r, u).start()
        return carry

    lax.fori_loop(0, ROWS_PER_W // NBUF, iter_body, 0)
    writeback(ROWS_PER_W - 2, 0).wait()
    writeback(ROWS_PER_W - 1, 1).wait()


@jax.jit
def kernel(indices, token_table, pos_table):
    idx_flat = indices.reshape(B * T)
    pos_flat = pos_table.reshape(T * D)
    table_pad = jnp.pad(token_table, ((0, 0), (0, D)))
    mesh = plsc.VectorSubcoreMesh(core_axis_name="c", subcore_axis_name="s")
    run = pl.kernel(
        _body,
        out_type=jax.ShapeDtypeStruct((B, T, D), jnp.float32),
        mesh=mesh,
        scratch_types=[
            pltpu.VMEM((T * D,), jnp.float32),
            pltpu.VMEM((ROWS_PER_W * T,), jnp.int32),
            pltpu.VMEM((NBUF, T, 2 * D), jnp.float32),
            pltpu.VMEM((NBUF, T, D), jnp.float32),
            pltpu.SemaphoreType.DMA,
            pltpu.SemaphoreType.DMA,
            pltpu.SemaphoreType.DMA,
            pltpu.SemaphoreType.DMA,
        ],
        compiler_params=pltpu.CompilerParams(use_tc_tiling_on_sc=False),
    )
    return run(idx_flat, table_pad, pos_flat)


# speedup vs baseline: 1.0492x; 1.0492x over previous
import jax
import jax.numpy as jnp
from jax import lax
from jax.experimental import pallas as pl
from jax.experimental.pallas import tpu as pltpu
from jax.experimental.pallas import tpu_sc as plsc

VOCAB = 100000
D = 64
T = 200
B = 1024

NC = 2
NS = 16
NW = NC * NS
ROWS_PER_W = B // NW
NBUF = 4
LANES = 16
EMBED_SCALE = 8.0

SPLITS = ((0, 104), (104, 96))


def _body(idx_hbm, table_hbm, pos_hbm, out_hbm,
          pos_v, idx_all, rows_v, out_v,
          sem_g0, sem_g1, sem_g2, sem_g3, sem_w0, sem_w1, sem_w2, sem_w3):
    wid = lax.axis_index("s") * NC + lax.axis_index("c")
    base = wid * ROWS_PER_W
    sem_g = (sem_g0, sem_g1, sem_g2, sem_g3)
    sem_w = (sem_w0, sem_w1, sem_w2, sem_w3)

    pltpu.sync_copy(pos_hbm, pos_v)
    pltpu.sync_copy(idx_hbm.at[pl.ds(base * T, ROWS_PER_W * T)], idx_all)

    def gather_half(r, u, h):
        off, ln = SPLITS[h]
        start = pl.multiple_of(r * T, 8) + off
        return pltpu.make_async_copy(
            table_hbm.at[idx_all.at[pl.ds(start, ln)]],
            rows_v.at[u, pl.ds(off, ln)], sem_g[u])

    def writeback(r, u):
        return pltpu.make_async_copy(out_v.at[u], out_hbm.at[base + r], sem_w[u])

    for u in range(NBUF):
        gather_half(u, u, 0).start()
        gather_half(u, u, 1).start()

    def iter_body(it, carry):
        for u in range(NBUF):
            r = it * NBUF + u
            gather_half(r, u, 0).wait()
            gather_half(r, u, 1).wait()

            @pl.when(it > 0)
            def _():
                writeback(r - NBUF, u).wait()

            @plsc.parallel_loop(0, T, 1, unroll=8)
            def _compute(i):
                for c in range(D // LANES):
                    sl = pl.ds(c * LANES, LANES)
                    out_v[u, i, sl] = (
                        rows_v[u, i, sl] * EMBED_SCALE
                        + pos_v[pl.ds(i * D + c * LANES, LANES)])

            @pl.when(it + 1 < ROWS_PER_W // NBUF)
            def _():
                gather_half(r + NBUF, u, 0).start()
                gather_half(r + NBUF, u, 1).start()

            writeback(r, u).start()
        return carry

    lax.fori_loop(0, ROWS_PER_W // NBUF, iter_body, 0)
    for u in range(NBUF):
        writeback(ROWS_PER_W - NBUF + u, u).wait()


@jax.jit
def kernel(indices, token_table, pos_table):
    idx_flat = indices.reshape(B * T)
    pos_flat = pos_table.reshape(T * D)
    mesh = plsc.VectorSubcoreMesh(core_axis_name="c", subcore_axis_name="s")
    run = pl.kernel(
        _body,
        out_type=jax.ShapeDtypeStruct((B, T, D), jnp.float32),
        mesh=mesh,
        scratch_types=[
            pltpu.VMEM((T * D,), jnp.float32),
            pltpu.VMEM((ROWS_PER_W * T,), jnp.int32),
            pltpu.VMEM((NBUF, T, D), jnp.float32),
            pltpu.VMEM((NBUF, T, D), jnp.float32),
        ] + [pltpu.SemaphoreType.DMA] * 8,
        compiler_params=pltpu.CompilerParams(use_tc_tiling_on_sc=False),
    )
    return run(idx_flat, token_table, pos_flat)
